# baseline (speedup 1.0000x reference)
import functools

import jax
import jax.numpy as jnp
from jax import lax
from jax.experimental import pallas as pl
from jax.experimental.pallas import tpu as pltpu
from jax.experimental.pallas import tpu_sc as plsc

HIDDEN = 64
NUM_CORES = 2
NUM_SUBCORES = 16
NUM_WORKERS = NUM_CORES * NUM_SUBCORES
CHUNK = 128


def _gather_kernel(b_per_w, n_chunks, idx_hbm, table_hbm, out_hbm,
                   idx_v, rows_v, sem):
  wid = lax.axis_index("s") * NUM_CORES + lax.axis_index("c")
  wbase = wid * b_per_w

  def body(i, carry):
    base = wbase + i * CHUNK
    pltpu.sync_copy(idx_hbm.at[pl.ds(base, CHUNK)], idx_v)
    pltpu.async_copy(table_hbm.at[idx_v], rows_v, sem).wait()
    pltpu.sync_copy(rows_v, out_hbm.at[pl.ds(base, CHUNK)])
    return carry

  lax.fori_loop(0, n_chunks, body, 0)


def _build_call(n_idx):
  assert n_idx % (NUM_WORKERS * CHUNK) == 0
  b_per_w = n_idx // NUM_WORKERS
  n_chunks = b_per_w // CHUNK
  mesh = plsc.VectorSubcoreMesh(core_axis_name="c", subcore_axis_name="s")
  return pl.kernel(
      functools.partial(_gather_kernel, b_per_w, n_chunks),
      out_type=jax.ShapeDtypeStruct((n_idx, HIDDEN), jnp.float32),
      mesh=mesh,
      scratch_types=[
          pltpu.VMEM((CHUNK,), jnp.int32),
          pltpu.VMEM((CHUNK, HIDDEN), jnp.float32),
          pltpu.SemaphoreType.DMA,
      ],
      compiler_params=pltpu.CompilerParams(use_tc_tiling_on_sc=False),
  )


@jax.jit
def kernel(input_ids, table):
  shape = input_ids.shape
  idx_flat = input_ids.reshape(-1).astype(jnp.int32)
  out = _build_call(idx_flat.shape[0])(idx_flat, table)
  return out.reshape(shape + (HIDDEN,))


# speedup vs baseline: 1.1895x; 1.1895x over previous
import functools

import jax
import jax.numpy as jnp
from jax import lax
from jax.experimental import pallas as pl
from jax.experimental.pallas import tpu as pltpu
from jax.experimental.pallas import tpu_sc as plsc

HIDDEN = 64
NUM_CORES = 2
NUM_SUBCORES = 16
NUM_WORKERS = NUM_CORES * NUM_SUBCORES
CHUNK = 128
SPC = 2
NBUF = 4
SLOT_ROWS = SPC * CHUNK


def _gather_kernel(n_chunks, idx_hbm, table_hbm, out_hbm,
                   idx_v, rows_v, gsems, wsems):
  n_groups = n_chunks // SPC
  n_rounds = n_groups // NBUF
  b_per_w = n_chunks * CHUNK
  wid = lax.axis_index("s") * NUM_CORES + lax.axis_index("c")
  wbase = wid * b_per_w

  pltpu.sync_copy(idx_hbm.at[pl.ds(wid * n_chunks, n_chunks)], idx_v)

  def gather_copy(g, s, j):
    return pltpu.make_async_copy(
        table_hbm.at[idx_v.at[g * SPC + j]],
        rows_v.at[s, pl.ds(j * CHUNK, CHUNK)],
        gsems[s])

  def write_copy(g, s):
    return pltpu.make_async_copy(
        rows_v.at[s],
        out_hbm.at[pl.ds(wbase + g * SLOT_ROWS, SLOT_ROWS)],
        wsems[s])

  for s in range(NBUF):
    for j in range(SPC):
      gather_copy(s, s, j).start()

  def body(r, carry):
    for s in range(NBUF):
      g = r * NBUF + s
      for j in range(SPC):
        gather_copy(g, s, j).wait()
      write_copy(g, s).start()
    for s in range(NBUF):
      g = r * NBUF + s
      write_copy(g, s).wait()
      for j in range(SPC):
        gather_copy(g + NBUF, s, j).start()
    return carry

  lax.fori_loop(0, n_rounds - 1, body, 0)

  r = n_rounds - 1
  for s in range(NBUF):
    g = r * NBUF + s
    for j in range(SPC):
      gather_copy(g, s, j).wait()
    write_copy(g, s).start()
  for s in range(NBUF):
    write_copy(r * NBUF + s, s).wait()


def _build_call(n_idx):
  assert n_idx % (NUM_WORKERS * CHUNK * SPC * NBUF) == 0
  n_chunks = n_idx // (NUM_WORKERS * CHUNK)
  mesh = plsc.VectorSubcoreMesh(core_axis_name="c", subcore_axis_name="s")
  return pl.kernel(
      functools.partial(_gather_kernel, n_chunks),
      out_type=jax.ShapeDtypeStruct((n_idx, HIDDEN), jnp.float32),
      mesh=mesh,
      scratch_types=[
          pltpu.VMEM((n_chunks, CHUNK), jnp.int32),
          pltpu.VMEM((NBUF, SLOT_ROWS, HIDDEN), jnp.float32),
          [pltpu.SemaphoreType.DMA] * NBUF,
          [pltpu.SemaphoreType.DMA] * NBUF,
      ],
      compiler_params=pltpu.CompilerParams(use_tc_tiling_on_sc=False),
  )


@jax.jit
def kernel(input_ids, table):
  shape = input_ids.shape
  idx_flat = input_ids.reshape(-1, CHUNK).astype(jnp.int32)
  out = _build_call(idx_flat.size)(idx_flat, table)
  return out.reshape(shape + (HIDDEN,))


# speedup vs baseline: 1.1923x; 1.0024x over previous
import functools

import jax
import jax.numpy as jnp
from jax import lax
from jax.experimental import pallas as pl
from jax.experimental.pallas import tpu as pltpu
from jax.experimental.pallas import tpu_sc as plsc

HIDDEN = 64
NUM_CORES = 2
NUM_SUBCORES = 16
NUM_WORKERS = NUM_CORES * NUM_SUBCORES
CHUNK = 128
NBUF = 10


def _gather_kernel(n_chunks, idx_hbm, table_hbm, out_hbm,
                   idx_v, rows_v, gsems, wsems):
  n_rounds = n_chunks // NBUF
  b_per_w = n_chunks * CHUNK
  wid = lax.axis_index("s") * NUM_CORES + lax.axis_index("c")
  wbase = wid * b_per_w

  pltpu.sync_copy(idx_hbm.at[pl.ds(wid * n_chunks, n_chunks)], idx_v)

  def gather_copy(g, s):
    return pltpu.make_async_copy(
        table_hbm.at[idx_v.at[g]],
        rows_v.at[s],
        gsems[s])

  def write_copy(g, s):
    return pltpu.make_async_copy(
        rows_v.at[s],
        out_hbm.at[pl.ds(wbase + g * CHUNK, CHUNK)],
        wsems[s])

  for s in range(NBUF):
    gather_copy(s, s).start()

  def body(r, carry):
    for s in range(NBUF):
      g = r * NBUF + s
      gather_copy(g, s).wait()
      write_copy(g, s).start()
    for s in range(NBUF):
      g = r * NBUF + s
      write_copy(g, s).wait()
      gather_copy(g + NBUF, s).start()
    return carry

  lax.fori_loop(0, n_rounds - 1, body, 0)

  r = n_rounds - 1
  for s in range(NBUF):
    g = r * NBUF + s
    gather_copy(g, s).wait()
    write_copy(g, s).start()
  for s in range(NBUF):
    write_copy(r * NBUF + s, s).wait()


def _build_call(n_idx):
  assert n_idx % (NUM_WORKERS * CHUNK * NBUF) == 0
  n_chunks = n_idx // (NUM_WORKERS * CHUNK)
  mesh = plsc.VectorSubcoreMesh(core_axis_name="c", subcore_axis_name="s")
  return pl.kernel(
      functools.partial(_gather_kernel, n_chunks),
      out_type=jax.ShapeDtypeStruct((n_idx, HIDDEN), jnp.float32),
      mesh=mesh,
      scratch_types=[
          pltpu.VMEM((n_chunks, CHUNK), jnp.int32),
          pltpu.VMEM((NBUF, CHUNK, HIDDEN), jnp.float32),
          [pltpu.SemaphoreType.DMA] * NBUF,
          [pltpu.SemaphoreType.DMA] * NBUF,
      ],
      compiler_params=pltpu.CompilerParams(use_tc_tiling_on_sc=False),
  )


@jax.jit
def kernel(input_ids, table):
  shape = input_ids.shape
  idx_flat = input_ids.reshape(-1, CHUNK).astype(jnp.int32)
  out = _build_call(idx_flat.size)(idx_flat, table)
  return out.reshape(shape + (HIDDEN,))
